# baseline (speedup 1.0000x reference)
import jax
import jax.numpy as jnp
from jax.experimental import pallas as pl

_BM = 200


def _gcn_block(adj0_ref, adj1_ref, x_ref, w_ref, b_ref, out_ref):
    x = x_ref[...]
    w = w_ref[...]
    b = b_ref[...]
    t0 = jnp.dot(adj0_ref[...], x, preferred_element_type=jnp.float32)
    out_ref[0:_BM, :] = jnp.dot(t0, w, preferred_element_type=jnp.float32) + b
    t1 = jnp.dot(adj1_ref[...], x, preferred_element_type=jnp.float32)
    out_ref[_BM:2 * _BM, :] = (
        jnp.dot(t1, w, preferred_element_type=jnp.float32) + b
    )


def kernel(x, adj, W, b):
    n, d_in = x.shape
    d_out = W.shape[1]
    b2 = b.reshape(1, d_out)
    grid = (n // (2 * _BM),)
    return pl.pallas_call(
        _gcn_block,
        grid=grid,
        in_specs=[
            pl.BlockSpec((_BM, n), lambda i: (2 * i, 0)),
            pl.BlockSpec((_BM, n), lambda i: (2 * i + 1, 0)),
            pl.BlockSpec((n, d_in), lambda i: (0, 0)),
            pl.BlockSpec((d_in, d_out), lambda i: (0, 0)),
            pl.BlockSpec((1, d_out), lambda i: (0, 0)),
        ],
        out_specs=pl.BlockSpec((2 * _BM, d_out), lambda i: (i, 0)),
        out_shape=jax.ShapeDtypeStruct((n, d_out), jnp.float32),
    )(adj, adj, x, W, b2)


# speedup vs baseline: 1.0920x; 1.0920x over previous
import jax
import jax.numpy as jnp
from jax.experimental import pallas as pl

_BM = 400


def _gcn_block(adj_ref, x_ref, w_ref, b_ref, out_ref):
    tmp = jnp.dot(adj_ref[...], x_ref[...], preferred_element_type=jnp.float32)
    out_ref[...] = (
        jnp.dot(tmp, w_ref[...], preferred_element_type=jnp.float32)
        + b_ref[...]
    )


def kernel(x, adj, W, b):
    n, d_in = x.shape
    d_out = W.shape[1]
    b2 = b.reshape(1, d_out)
    grid = (n // _BM,)
    return pl.pallas_call(
        _gcn_block,
        grid=grid,
        in_specs=[
            pl.BlockSpec((_BM, n), lambda i: (i, 0)),
            pl.BlockSpec((n, d_in), lambda i: (0, 0)),
            pl.BlockSpec((d_in, d_out), lambda i: (0, 0)),
            pl.BlockSpec((1, d_out), lambda i: (0, 0)),
        ],
        out_specs=pl.BlockSpec((_BM, d_out), lambda i: (i, 0)),
        out_shape=jax.ShapeDtypeStruct((n, d_out), jnp.float32),
    )(adj, x, W, b2)
